# Initial kernel scaffold; baseline (speedup 1.0000x reference)
import jax
import jax.numpy as jnp
from jax.experimental import pallas as pl


def kernel(points, ray_o, ray_d):
    raise NotImplementedError("write your pallas kernel here")


# speedup vs baseline: 1.9092x; 1.9092x over previous
import functools

import jax
import jax.numpy as jnp
from jax import lax
from jax.experimental import pallas as pl

R_TILE = 256
C_BLK = 2048
KC = 8
BIG = 1.0e30
IBIG = 2**30


def _knn_body(n_real, n_pad, ro_ref, rd_ref, pts_ref, dist_out, t_out, idx_out):
    ox = ro_ref[:, 0:1]
    oy = ro_ref[:, 1:2]
    oz = ro_ref[:, 2:3]
    dx = rd_ref[:, 0:1]
    dy = rd_ref[:, 1:2]
    dz = rd_ref[:, 2:3]
    inv = 1.0 / jnp.sqrt(dx * dx + dy * dy + dz * dz + 1e-12)
    dx = dx * inv
    dy = dy * inv
    dz = dz * inv

    col0 = lax.broadcasted_iota(jnp.int32, (R_TILE, C_BLK), 1)
    lane128 = lax.broadcasted_iota(jnp.int32, (R_TILE, 128), 1)
    nblk = n_pad // C_BLK

    def blk(b, carry):
        cv, ct, ci = carry
        px = pts_ref[0:1, pl.ds(b * C_BLK, C_BLK)]
        py = pts_ref[1:2, pl.ds(b * C_BLK, C_BLK)]
        pz = pts_ref[2:3, pl.ds(b * C_BLK, C_BLK)]
        ax = px - ox
        ay = py - oy
        az = pz - oz
        t = ax * dx + ay * dy + az * dz
        rx = ax - t * dx
        ry = ay - t * dy
        rz = az - t * dz
        d2 = rx * rx + ry * ry + rz * rz
        d = jnp.sqrt(jnp.maximum(d2, 1e-12))
        col = col0 + b * C_BLK
        d = jnp.where(col < n_real, d, BIG)

        ext_d = jnp.concatenate([cv, d], axis=1)
        ext_t = jnp.concatenate([ct, t], axis=1)
        ext_c = jnp.concatenate([ci, col], axis=1)

        nv = jnp.full((R_TILE, 128), BIG, jnp.float32)
        nt = jnp.zeros((R_TILE, 128), jnp.float32)
        nc = jnp.full((R_TILE, 128), IBIG, jnp.int32)
        for k in range(KC):
            m = jnp.min(ext_d, axis=1, keepdims=True)
            a = jnp.min(jnp.where(ext_d == m, ext_c, IBIG), axis=1, keepdims=True)
            sel = ext_c == a
            tk = jnp.sum(jnp.where(sel, ext_t, 0.0), axis=1, keepdims=True)
            ext_d = jnp.where(sel, BIG, ext_d)
            nv = jnp.where(lane128 == k, m, nv)
            nt = jnp.where(lane128 == k, tk, nt)
            nc = jnp.where(lane128 == k, a, nc)
        return nv, nt, nc

    cv0 = jnp.full((R_TILE, 128), BIG, jnp.float32)
    ct0 = jnp.zeros((R_TILE, 128), jnp.float32)
    ci0 = jnp.full((R_TILE, 128), IBIG, jnp.int32)
    cv, ct, ci = lax.fori_loop(0, nblk, blk, (cv0, ct0, ci0))
    dist_out[...] = cv[:, 0:KC]
    t_out[...] = ct[:, 0:KC]
    idx_out[...] = ci[:, 0:KC]


def kernel(points, ray_o, ray_d):
    n_real = points.shape[0]
    n_rays = ray_o.shape[0]
    n_pad = ((n_real + C_BLK - 1) // C_BLK) * C_BLK
    pts_t = jnp.pad(points.T, ((0, 0), (0, n_pad - n_real)))

    grid = (n_rays // R_TILE,)
    out_shapes = (
        jax.ShapeDtypeStruct((n_rays, KC), jnp.float32),
        jax.ShapeDtypeStruct((n_rays, KC), jnp.float32),
        jax.ShapeDtypeStruct((n_rays, KC), jnp.int32),
    )
    body = functools.partial(_knn_body, n_real, n_pad)
    return pl.pallas_call(
        body,
        grid=grid,
        in_specs=[
            pl.BlockSpec((R_TILE, 3), lambda i: (i, 0)),
            pl.BlockSpec((R_TILE, 3), lambda i: (i, 0)),
            pl.BlockSpec((3, n_pad), lambda i: (0, 0)),
        ],
        out_specs=(
            pl.BlockSpec((R_TILE, KC), lambda i: (i, 0)),
            pl.BlockSpec((R_TILE, KC), lambda i: (i, 0)),
            pl.BlockSpec((R_TILE, KC), lambda i: (i, 0)),
        ),
        out_shape=out_shapes,
    )(ray_o, ray_d, pts_t)


# speedup vs baseline: 5.8132x; 3.0449x over previous
import functools

import jax
import jax.numpy as jnp
from jax import lax
from jax.experimental import pallas as pl
from jax.experimental.pallas import tpu as pltpu
from jax.experimental.pallas import tpu_sc as plsc

R_TILE = 256
C_BLK = 2048
KC = 8
BIG = 1.0e30
IBIG = 2**30


def _ray_basis(ro_ref, rd_ref):
    ox = ro_ref[:, 0:1]
    oy = ro_ref[:, 1:2]
    oz = ro_ref[:, 2:3]
    dx = rd_ref[:, 0:1]
    dy = rd_ref[:, 1:2]
    dz = rd_ref[:, 2:3]
    inv = 1.0 / jnp.sqrt(dx * dx + dy * dy + dz * dz + 1e-12)
    return ox, oy, oz, dx * inv, dy * inv, dz * inv


def _group_body(n_real, n_pad, ro_ref, rd_ref, pts_ref, gsel_ref, msc):
    ox, oy, oz, dx, dy, dz = _ray_basis(ro_ref, rd_ref)
    col0 = lax.broadcasted_iota(jnp.int32, (R_TILE, C_BLK), 1)
    nblk = n_pad // C_BLK

    def blk(b, _):
        px = pts_ref[0:1, pl.ds(b * C_BLK, C_BLK)]
        py = pts_ref[1:2, pl.ds(b * C_BLK, C_BLK)]
        pz = pts_ref[2:3, pl.ds(b * C_BLK, C_BLK)]
        ax = px - ox
        ay = py - oy
        az = pz - oz
        t = ax * dx + ay * dy + az * dz
        rx = ax - t * dx
        ry = ay - t * dy
        rz = az - t * dz
        d2 = rx * rx + ry * ry + rz * rz
        col = col0 + b * C_BLK
        d2 = jnp.where(col < n_real, d2, BIG)
        m = d2[:, 0:128]
        for k in range(1, C_BLK // 128):
            m = jnp.minimum(m, d2[:, k * 128:(k + 1) * 128])
        msc[:, pl.ds(b * 128, 128)] = m
        return 0

    lax.fori_loop(0, nblk, blk, 0)

    ng = nblk * 128
    M = msc[...]
    glane = lax.broadcasted_iota(jnp.int32, (R_TILE, ng), 1)
    lane128 = lax.broadcasted_iota(jnp.int32, (R_TILE, 128), 1)
    gout = jnp.full((R_TILE, 128), IBIG, jnp.int32)
    for k in range(KC):
        m = jnp.min(M, axis=1, keepdims=True)
        g = jnp.min(jnp.where(M == m, glane, IBIG), axis=1, keepdims=True)
        M = jnp.where(glane == g, BIG, M)
        gout = jnp.where(lane128 == k, g, gout)
    gsel_ref[...] = gout[:, 0:KC]


def _cand_body(n_real, ro_ref, rd_ref, px_ref, py_ref, pz_ref, grep_ref,
               dist_out, t_out, idx_out):
    ox, oy, oz, dx, dy, dz = _ray_basis(ro_ref, rd_ref)
    gl = grep_ref[...]
    lane = lax.broadcasted_iota(jnp.int32, (R_TILE, 128), 1)
    pid = (gl // 128) * C_BLK + (gl % 128) + (lane % 16) * 128
    px = px_ref[...]
    py = py_ref[...]
    pz = pz_ref[...]
    ax = px - ox
    ay = py - oy
    az = pz - oz
    t = ax * dx + ay * dy + az * dz
    rx = ax - t * dx
    ry = ay - t * dy
    rz = az - t * dz
    d2 = rx * rx + ry * ry + rz * rz
    d = jnp.sqrt(jnp.maximum(d2, 1e-12))
    d = jnp.where(pid < n_real, d, BIG)

    nv = jnp.full((R_TILE, 128), BIG, jnp.float32)
    nt = jnp.zeros((R_TILE, 128), jnp.float32)
    nc = jnp.full((R_TILE, 128), IBIG, jnp.int32)
    for k in range(KC):
        m = jnp.min(d, axis=1, keepdims=True)
        a = jnp.min(jnp.where(d == m, pid, IBIG), axis=1, keepdims=True)
        sel = pid == a
        tk = jnp.sum(jnp.where(sel, t, 0.0), axis=1, keepdims=True)
        d = jnp.where(sel, BIG, d)
        nv = jnp.where(lane == k, m, nv)
        nt = jnp.where(lane == k, tk, nt)
        nc = jnp.where(lane == k, a, nc)
    dist_out[...] = nv[:, 0:KC]
    t_out[...] = nt[:, 0:KC]
    idx_out[...] = nc[:, 0:KC]


def _gather_groups(table, idxf):
    B = idxf.shape[0]
    D = table.shape[1]
    info = plsc.get_sparse_core_info()
    NC, NS = info.num_cores, info.num_subcores
    NW = NC * NS
    nrow = B // 128
    rpw = nrow // NW
    idx2 = idxf.reshape(nrow, 128)
    mesh = plsc.VectorSubcoreMesh(core_axis_name="c", subcore_axis_name="s")

    @functools.partial(
        pl.kernel, mesh=mesh,
        out_type=jax.ShapeDtypeStruct((nrow, 128, D), jnp.float32),
        scratch_types=[
            pltpu.VMEM((rpw, 128), jnp.int32),
            pltpu.VMEM((rpw, 128, D), jnp.float32),
            pltpu.SemaphoreType.DMA,
        ],
    )
    def gk(table_hbm, idx_hbm, out_hbm, idx_v, rows_v, sem):
        wid = lax.axis_index("s") * NC + lax.axis_index("c")
        base = wid * rpw
        pltpu.sync_copy(idx_hbm.at[pl.ds(base, rpw)], idx_v)
        copies = [
            pltpu.async_copy(table_hbm.at[idx_v.at[j]], rows_v.at[j], sem)
            for j in range(rpw)
        ]
        for c in copies:
            c.wait()
        pltpu.sync_copy(rows_v, out_hbm.at[pl.ds(base, rpw)])

    return gk(table, idx2).reshape(B, D)


def kernel(points, ray_o, ray_d):
    n_real = points.shape[0]
    n_rays = ray_o.shape[0]
    nblk = (n_real + C_BLK - 1) // C_BLK
    n_pad = nblk * C_BLK
    ng = nblk * 128

    pts_t = jnp.pad(points.T, ((0, 0), (0, n_pad - n_real)))
    pts_pad = jnp.pad(points, ((0, n_pad - n_real), (0, 0)))
    tbl = pts_pad.reshape(nblk, 16, 128, 3).transpose(0, 2, 1, 3).reshape(ng, 48)
    tbl = jnp.pad(tbl, ((0, 0), (0, 128 - 48)))

    grid = (n_rays // R_TILE,)
    gsel = pl.pallas_call(
        functools.partial(_group_body, n_real, n_pad),
        grid=grid,
        in_specs=[
            pl.BlockSpec((R_TILE, 3), lambda i: (i, 0)),
            pl.BlockSpec((R_TILE, 3), lambda i: (i, 0)),
            pl.BlockSpec((3, n_pad), lambda i: (0, 0)),
        ],
        out_specs=pl.BlockSpec((R_TILE, KC), lambda i: (i, 0)),
        out_shape=jax.ShapeDtypeStruct((n_rays, KC), jnp.int32),
        scratch_shapes=[pltpu.VMEM((R_TILE, ng), jnp.float32)],
    )(ray_o, ray_d, pts_t)

    gathered = _gather_groups(tbl, gsel.reshape(n_rays * KC))[:, 0:48]
    pxyz = gathered.reshape(n_rays, KC, 16, 3).transpose(0, 3, 1, 2)
    pxyz = pxyz.reshape(n_rays, 3 * 128)
    px = pxyz[:, 0:128]
    py = pxyz[:, 128:256]
    pz = pxyz[:, 256:384]
    grep = jnp.broadcast_to(gsel[:, :, None], (n_rays, KC, 16)).reshape(n_rays, 128)

    out_shapes = (
        jax.ShapeDtypeStruct((n_rays, KC), jnp.float32),
        jax.ShapeDtypeStruct((n_rays, KC), jnp.float32),
        jax.ShapeDtypeStruct((n_rays, KC), jnp.int32),
    )
    return pl.pallas_call(
        functools.partial(_cand_body, n_real),
        grid=grid,
        in_specs=[
            pl.BlockSpec((R_TILE, 3), lambda i: (i, 0)),
            pl.BlockSpec((R_TILE, 3), lambda i: (i, 0)),
            pl.BlockSpec((R_TILE, 128), lambda i: (i, 0)),
            pl.BlockSpec((R_TILE, 128), lambda i: (i, 0)),
            pl.BlockSpec((R_TILE, 128), lambda i: (i, 0)),
            pl.BlockSpec((R_TILE, 128), lambda i: (i, 0)),
        ],
        out_specs=(
            pl.BlockSpec((R_TILE, KC), lambda i: (i, 0)),
            pl.BlockSpec((R_TILE, KC), lambda i: (i, 0)),
            pl.BlockSpec((R_TILE, KC), lambda i: (i, 0)),
        ),
        out_shape=out_shapes,
    )(ray_o, ray_d, px, py, pz, grep)
